# Initial kernel scaffold; baseline (speedup 1.0000x reference)
import jax
import jax.numpy as jnp
from jax.experimental import pallas as pl


def kernel(x, edge_index, batch, W1, a_src1, a_dst1, b1, g1, be1, W2, a_src2, a_dst2, b2, g2, be2, fc_w, fc_b, fc1_w, fc1_b):
    raise NotImplementedError("write your pallas kernel here")


# speedup vs baseline: 6.9963x; 6.9963x over previous
import functools

import jax
import jax.numpy as jnp
from jax import lax
from jax.experimental import pallas as pl
from jax.experimental.pallas import tpu as pltpu
from jax.experimental.pallas import tpu_sc as plsc

N = 10000
E = 320000
E2 = E + N
D_IN = 128
H = 512
HD = H + 16
FC = 256
OUT = 10
NG = 64

NW = 32
EPAD = ((E2 + NW * 16 - 1) // (NW * 16)) * (NW * 16)
EW = EPAD // NW
CHUNK = 8
BN = 1000
SENT = 1 << 30


def _f16(val, dtype=jnp.float32):
    return jnp.full((16,), val, dtype=dtype)


def _sc_edge_kernel(src_hbm, dst_hbm, asrc_hbm, adst_hbm, c_hbm, h_hbm,
                    agg_hbm, sidef_hbm, sidel_hbm, ids_hbm,
                    sv, dv, exv, asv, adv, cv, rows, acc, zrow, idsv, sem):
    wid = lax.axis_index("s") * 2 + lax.axis_index("c")
    base = wid * EW

    pltpu.sync_copy(src_hbm.at[pl.ds(base, EW)], sv)
    pltpu.sync_copy(dst_hbm.at[pl.ds(base, EW)], dv)
    pltpu.sync_copy(asrc_hbm, asv)
    pltpu.sync_copy(adst_hbm, adv)
    pltpu.sync_copy(c_hbm, cv)

    zero16 = jnp.zeros((16,), jnp.float32)
    for j in range(HD // 16):
        acc[pl.ds(j * 16, 16)] = zero16
        zrow[pl.ds(j * 16, 16)] = zero16

    cvec = cv[...]
    iota16 = lax.iota(jnp.int32, 16)

    first_d = jnp.max(plsc.load_gather(dv, [_f16(0, jnp.int32)]))
    last_d = jnp.max(plsc.load_gather(dv, [_f16(EW - 1, jnp.int32)]))
    pltpu.sync_copy(zrow, agg_hbm.at[first_d])
    pltpu.sync_copy(zrow, agg_hbm.at[last_d])

    def phase_a(i, carry):
        off = pl.multiple_of(i * 16, 16)
        s16 = sv[pl.ds(off, 16)]
        d16 = dv[pl.ds(off, 16)]
        asg = plsc.load_gather(asv, [s16])
        adg = plsc.load_gather(adv, [d16])
        s = asg + adg
        e = jnp.where(s >= 0.0, s, 0.2 * s)
        ex = jnp.exp(e - cvec)
        gidx = _f16(base + i * 16, jnp.int32) + iota16
        ex = jnp.where(gidx < E2, ex, 0.0)
        exv[pl.ds(off, 16)] = ex
        return carry

    lax.fori_loop(0, EW // 16, phase_a, 0)

    def phase_b(cidx, carry):
        prev, seg = carry
        off = pl.multiple_of(cidx * CHUNK, 8)
        pltpu.async_copy(h_hbm.at[sv.at[pl.ds(off, CHUNK)]], rows, sem).wait()
        for j in range(CHUNK):
            jb = _f16(0, jnp.int32) + (off + j)
            exb = plsc.load_gather(exv, [jb])
            dsc = jnp.max(plsc.load_gather(dv, [jb]))
            changed = dsc != prev

            @pl.when(jnp.logical_and(changed, seg > 0))
            def _flush_interior():
                pltpu.sync_copy(acc, agg_hbm.at[prev])

            @pl.when(jnp.logical_and(changed, seg == 0))
            def _flush_first():
                pltpu.sync_copy(acc, sidef_hbm.at[wid])

            @pl.when(changed)
            def _clear():
                for j2 in range(HD // 16):
                    acc[pl.ds(j2 * 16, 16)] = zero16

            for j2 in range(H // 16):
                sl = pl.ds(j2 * 16, 16)
                acc[sl] += exb * rows[j, sl]
            acc[pl.ds(H, 16)] += exb
            prev = dsc
            seg = seg + changed.astype(jnp.int32)
        return prev, seg

    prev_f, seg_f = lax.fori_loop(0, EW // CHUNK, phase_b,
                                  (first_d, jnp.int32(0)))

    pltpu.sync_copy(acc, sidel_hbm.at[wid])

    @pl.when(seg_f == 0)
    def _no_first():
        pltpu.sync_copy(zrow, sidef_hbm.at[wid])

    fd_out = jnp.where(seg_f == 0, jnp.int32(SENT), first_d)
    ids = jnp.where(iota16 == 0, _f16(0, jnp.int32) + fd_out,
                    jnp.where(iota16 == 1, _f16(0, jnp.int32) + prev_f,
                              _f16(SENT, jnp.int32)))
    idsv[...] = ids
    pltpu.sync_copy(idsv, ids_hbm.at[wid])


def _sc_edge_call(src_p, dst_p, asrc, adst, cvec, h):
    mesh = plsc.VectorSubcoreMesh(core_axis_name="c", subcore_axis_name="s")
    fn = functools.partial(
        pl.kernel, mesh=mesh,
        compiler_params=pltpu.CompilerParams(needs_layout_passes=False),
        out_type=[
            jax.ShapeDtypeStruct((N, HD), jnp.float32),
            jax.ShapeDtypeStruct((NW, HD), jnp.float32),
            jax.ShapeDtypeStruct((NW, HD), jnp.float32),
            jax.ShapeDtypeStruct((NW, 16), jnp.int32),
        ],
        scratch_types=[
            pltpu.VMEM((EW,), jnp.int32),
            pltpu.VMEM((EW,), jnp.int32),
            pltpu.VMEM((EW,), jnp.float32),
            pltpu.VMEM((N,), jnp.float32),
            pltpu.VMEM((N,), jnp.float32),
            pltpu.VMEM((16,), jnp.float32),
            pltpu.VMEM((CHUNK, H), jnp.float32),
            pltpu.VMEM((HD,), jnp.float32),
            pltpu.VMEM((HD,), jnp.float32),
            pltpu.VMEM((16,), jnp.int32),
            pltpu.SemaphoreType.DMA,
        ],
    )(_sc_edge_kernel)
    return fn(src_p, dst_p, asrc, adst, cvec, h)


def _pre1_body(x_ref, w_ref, as_ref, ad_ref, h_ref, s_ref, d_ref, ms_ref,
               md_ref):
    i = pl.program_id(0)
    h = jnp.dot(x_ref[...], w_ref[...], preferred_element_type=jnp.float32)
    h_ref[...] = h
    s = jnp.dot(h, as_ref[...], preferred_element_type=jnp.float32)
    d = jnp.dot(h, ad_ref[...], preferred_element_type=jnp.float32)
    s_ref[...] = s
    d_ref[...] = d

    @pl.when(i == 0)
    def _init():
        ms_ref[...] = jnp.full((1, 1), -jnp.inf, jnp.float32)
        md_ref[...] = jnp.full((1, 1), -jnp.inf, jnp.float32)

    ms_ref[...] = jnp.maximum(ms_ref[...], jnp.max(s))
    md_ref[...] = jnp.maximum(md_ref[...], jnp.max(d))


def _pre1_call(x, W1, a_src, a_dst):
    grid = (N // BN,)
    return pl.pallas_call(
        _pre1_body,
        grid=grid,
        in_specs=[
            pl.BlockSpec((BN, D_IN), lambda i: (i, 0)),
            pl.BlockSpec((D_IN, H), lambda i: (0, 0)),
            pl.BlockSpec((H, 1), lambda i: (0, 0)),
            pl.BlockSpec((H, 1), lambda i: (0, 0)),
        ],
        out_specs=[
            pl.BlockSpec((BN, H), lambda i: (i, 0)),
            pl.BlockSpec((BN, 1), lambda i: (i, 0)),
            pl.BlockSpec((BN, 1), lambda i: (i, 0)),
            pl.BlockSpec((1, 1), lambda i: (0, 0)),
            pl.BlockSpec((1, 1), lambda i: (0, 0)),
        ],
        out_shape=[
            jax.ShapeDtypeStruct((N, H), jnp.float32),
            jax.ShapeDtypeStruct((N, 1), jnp.float32),
            jax.ShapeDtypeStruct((N, 1), jnp.float32),
            jax.ShapeDtypeStruct((1, 1), jnp.float32),
            jax.ShapeDtypeStruct((1, 1), jnp.float32),
        ],
    )(x, W1, a_src, a_dst)


def _fix_body(agg_ref, sf_ref, sl_ref, ids_ref, b_ref, out_ref, sm_ref,
              sq_ref):
    i = pl.program_id(0)
    rowid = i * BN + lax.broadcasted_iota(jnp.int32, (BN, 1), 0)
    ids = ids_ref[...]
    idf = ids[:, 0].reshape(1, NW)
    idl = ids[:, 1].reshape(1, NW)
    eqf = (rowid == idf).astype(jnp.float32)
    eql = (rowid == idl).astype(jnp.float32)
    fix = (agg_ref[...]
           + jnp.dot(eqf, sf_ref[...], preferred_element_type=jnp.float32)
           + jnp.dot(eql, sl_ref[...], preferred_element_type=jnp.float32))
    row = fix[:, :H] / (fix[:, H:H + 1] + 1e-16) + b_ref[...]
    out_ref[...] = row

    @pl.when(i == 0)
    def _init():
        sm_ref[...] = jnp.zeros((1, H), jnp.float32)
        sq_ref[...] = jnp.zeros((1, H), jnp.float32)

    sm_ref[...] += jnp.sum(row, axis=0, keepdims=True)
    sq_ref[...] += jnp.sum(row * row, axis=0, keepdims=True)


def _fix_call(agg, sidef, sidel, ids, b):
    grid = (N // BN,)
    return pl.pallas_call(
        _fix_body,
        grid=grid,
        in_specs=[
            pl.BlockSpec((BN, HD), lambda i: (i, 0)),
            pl.BlockSpec((NW, HD), lambda i: (0, 0)),
            pl.BlockSpec((NW, HD), lambda i: (0, 0)),
            pl.BlockSpec((NW, 16), lambda i: (0, 0)),
            pl.BlockSpec((1, H), lambda i: (0, 0)),
        ],
        out_specs=[
            pl.BlockSpec((BN, H), lambda i: (i, 0)),
            pl.BlockSpec((1, H), lambda i: (0, 0)),
            pl.BlockSpec((1, H), lambda i: (0, 0)),
        ],
        out_shape=[
            jax.ShapeDtypeStruct((N, H), jnp.float32),
            jax.ShapeDtypeStruct((1, H), jnp.float32),
            jax.ShapeDtypeStruct((1, H), jnp.float32),
        ],
    )(agg, sidef, sidel, ids, b)


def _norm_pre2_body(x_ref, sm_ref, sq_ref, g_ref, be_ref, bat_ref, w_ref,
                    as_ref, ad_ref, hr_ref, h2_ref, s_ref, d_ref, ms_ref,
                    md_ref, pool_ref, cnt_ref):
    i = pl.program_id(0)
    mu = sm_ref[...] / N
    var = sq_ref[...] / N - mu * mu
    xn = (x_ref[...] - mu) / jnp.sqrt(var + 1e-5) * g_ref[...] + be_ref[...]
    hr = jnp.maximum(xn, 0.0)
    hr_ref[...] = hr
    h2 = jnp.dot(hr, w_ref[...], preferred_element_type=jnp.float32)
    h2_ref[...] = h2
    s = jnp.dot(h2, as_ref[...], preferred_element_type=jnp.float32)
    d = jnp.dot(h2, ad_ref[...], preferred_element_type=jnp.float32)
    s_ref[...] = s
    d_ref[...] = d

    grp = lax.broadcasted_iota(jnp.int32, (NG, BN), 0)
    bo = (grp == bat_ref[...].reshape(1, BN)).astype(jnp.float32)

    @pl.when(i == 0)
    def _init():
        ms_ref[...] = jnp.full((1, 1), -jnp.inf, jnp.float32)
        md_ref[...] = jnp.full((1, 1), -jnp.inf, jnp.float32)
        pool_ref[...] = jnp.zeros((NG, H), jnp.float32)
        cnt_ref[...] = jnp.zeros((NG, 1), jnp.float32)

    ms_ref[...] = jnp.maximum(ms_ref[...], jnp.max(s))
    md_ref[...] = jnp.maximum(md_ref[...], jnp.max(d))
    pool_ref[...] += jnp.dot(bo, hr, preferred_element_type=jnp.float32)
    cnt_ref[...] += jnp.sum(bo, axis=1, keepdims=True)


def _norm_pre2_call(x, sm, sq, g, be, bat, W2, a_src, a_dst):
    grid = (N // BN,)
    return pl.pallas_call(
        _norm_pre2_body,
        grid=grid,
        in_specs=[
            pl.BlockSpec((BN, H), lambda i: (i, 0)),
            pl.BlockSpec((1, H), lambda i: (0, 0)),
            pl.BlockSpec((1, H), lambda i: (0, 0)),
            pl.BlockSpec((1, H), lambda i: (0, 0)),
            pl.BlockSpec((1, H), lambda i: (0, 0)),
            pl.BlockSpec((BN, 1), lambda i: (i, 0)),
            pl.BlockSpec((H, H), lambda i: (0, 0)),
            pl.BlockSpec((H, 1), lambda i: (0, 0)),
            pl.BlockSpec((H, 1), lambda i: (0, 0)),
        ],
        out_specs=[
            pl.BlockSpec((BN, H), lambda i: (i, 0)),
            pl.BlockSpec((BN, H), lambda i: (i, 0)),
            pl.BlockSpec((BN, 1), lambda i: (i, 0)),
            pl.BlockSpec((BN, 1), lambda i: (i, 0)),
            pl.BlockSpec((1, 1), lambda i: (0, 0)),
            pl.BlockSpec((1, 1), lambda i: (0, 0)),
            pl.BlockSpec((NG, H), lambda i: (0, 0)),
            pl.BlockSpec((NG, 1), lambda i: (0, 0)),
        ],
        out_shape=[
            jax.ShapeDtypeStruct((N, H), jnp.float32),
            jax.ShapeDtypeStruct((N, H), jnp.float32),
            jax.ShapeDtypeStruct((N, 1), jnp.float32),
            jax.ShapeDtypeStruct((N, 1), jnp.float32),
            jax.ShapeDtypeStruct((1, 1), jnp.float32),
            jax.ShapeDtypeStruct((1, 1), jnp.float32),
            jax.ShapeDtypeStruct((NG, H), jnp.float32),
            jax.ShapeDtypeStruct((NG, 1), jnp.float32),
        ],
    )(x, sm, sq, g, be, bat, W2, a_src, a_dst)


def _final_body(x_ref, sm_ref, sq_ref, g_ref, be_ref, bat_ref, p1_ref,
                cnt_ref, fcw_ref, fcb_ref, fc1w_ref, fc1b_ref, p2_ref,
                out_ref):
    i = pl.program_id(0)
    mu = sm_ref[...] / N
    var = sq_ref[...] / N - mu * mu
    xn = (x_ref[...] - mu) / jnp.sqrt(var + 1e-5) * g_ref[...] + be_ref[...]
    hr = jnp.maximum(xn, 0.0)

    grp = lax.broadcasted_iota(jnp.int32, (NG, BN), 0)
    bo = (grp == bat_ref[...].reshape(1, BN)).astype(jnp.float32)

    @pl.when(i == 0)
    def _init():
        p2_ref[...] = jnp.zeros((NG, H), jnp.float32)

    p2_ref[...] += jnp.dot(bo, hr, preferred_element_type=jnp.float32)

    @pl.when(i == (N // BN) - 1)
    def _head():
        cnt = jnp.maximum(cnt_ref[...], 1.0)
        z = p1_ref[...] / cnt + p2_ref[...] / cnt
        z = jnp.maximum(
            jnp.dot(z, fcw_ref[...], preferred_element_type=jnp.float32)
            + fcb_ref[...], 0.0)
        z = jnp.dot(z, fc1w_ref[...],
                    preferred_element_type=jnp.float32) + fc1b_ref[...]
        m = jnp.max(z, axis=1, keepdims=True)
        lse = m + jnp.log(jnp.sum(jnp.exp(z - m), axis=1, keepdims=True))
        out_ref[...] = z - lse


def _final_call(x, sm, sq, g, be, bat, pool1, cnt, fcw, fcb, fc1w, fc1b):
    grid = (N // BN,)
    return pl.pallas_call(
        _final_body,
        grid=grid,
        in_specs=[
            pl.BlockSpec((BN, H), lambda i: (i, 0)),
            pl.BlockSpec((1, H), lambda i: (0, 0)),
            pl.BlockSpec((1, H), lambda i: (0, 0)),
            pl.BlockSpec((1, H), lambda i: (0, 0)),
            pl.BlockSpec((1, H), lambda i: (0, 0)),
            pl.BlockSpec((BN, 1), lambda i: (i, 0)),
            pl.BlockSpec((NG, H), lambda i: (0, 0)),
            pl.BlockSpec((NG, 1), lambda i: (0, 0)),
            pl.BlockSpec((H, FC), lambda i: (0, 0)),
            pl.BlockSpec((1, FC), lambda i: (0, 0)),
            pl.BlockSpec((FC, OUT), lambda i: (0, 0)),
            pl.BlockSpec((1, OUT), lambda i: (0, 0)),
        ],
        out_specs=[
            pl.BlockSpec((NG, H), lambda i: (0, 0)),
            pl.BlockSpec((NG, OUT), lambda i: (0, 0)),
        ],
        out_shape=[
            jax.ShapeDtypeStruct((NG, H), jnp.float32),
            jax.ShapeDtypeStruct((NG, OUT), jnp.float32),
        ],
    )(x, sm, sq, g, be, bat, pool1, cnt, fcw, fcb, fc1w, fc1b)


def _shift_const(ms, md):
    c = ms[0, 0] + md[0, 0]
    c = jnp.where(c > 0.0, c, 0.2 * c)
    return jnp.full((16,), c, jnp.float32)


def kernel(x, edge_index, batch, W1, a_src1, a_dst1, b1, g1, be1,
           W2, a_src2, a_dst2, b2, g2, be2, fc_w, fc_b, fc1_w, fc1_b):
    loop = jnp.arange(N, dtype=jnp.int32)
    src = jnp.concatenate([edge_index[0].astype(jnp.int32), loop])
    dst = jnp.concatenate([edge_index[1].astype(jnp.int32), loop])
    order = jnp.argsort(dst)
    src_s = src[order]
    dst_s = dst[order]
    npad = EPAD - E2
    src_p = jnp.concatenate([src_s, jnp.zeros((npad,), jnp.int32)])
    dst_p = jnp.concatenate([dst_s, jnp.full((npad,), N - 1, jnp.int32)])
    bat = batch.astype(jnp.int32).reshape(N, 1)

    h1, s1, d1, ms1, md1 = _pre1_call(
        x, W1, a_src1.reshape(H, 1), a_dst1.reshape(H, 1))
    c1 = _shift_const(ms1, md1)
    agg1, sf1, sl1, ids1 = _sc_edge_call(
        src_p, dst_p, s1.reshape(N), d1.reshape(N), c1, h1)
    out1, sm1, sq1 = _fix_call(agg1, sf1, sl1, ids1, b1.reshape(1, H))

    hr1, h2, s2, d2, ms2, md2, pool1, cnt = _norm_pre2_call(
        out1, sm1, sq1, g1.reshape(1, H), be1.reshape(1, H), bat, W2,
        a_src2.reshape(H, 1), a_dst2.reshape(H, 1))

    c2 = _shift_const(ms2, md2)
    agg2, sf2, sl2, ids2 = _sc_edge_call(
        src_p, dst_p, s2.reshape(N), d2.reshape(N), c2, h2)
    out2, sm2, sq2 = _fix_call(agg2, sf2, sl2, ids2, b2.reshape(1, H))

    _, out = _final_call(
        out2, sm2, sq2, g2.reshape(1, H), be2.reshape(1, H), bat, pool1,
        cnt, fc_w, fc_b.reshape(1, FC), fc1_w, fc1_b.reshape(1, OUT))
    return out


# speedup vs baseline: 7.7285x; 1.1047x over previous
import functools

import jax
import jax.numpy as jnp
from jax import lax
from jax.experimental import pallas as pl
from jax.experimental.pallas import tpu as pltpu
from jax.experimental.pallas import tpu_sc as plsc

N = 10000
E = 320000
E2 = E + N
D_IN = 128
H = 512
HD = H + 16
FC = 256
OUT = 10
NG = 64

NW = 32
EPAD = ((E2 + NW * 16 - 1) // (NW * 16)) * (NW * 16)
EW = EPAD // NW
CHUNK = 8
BN = 1000
SENT = 1 << 30


def _f16(val, dtype=jnp.float32):
    return jnp.full((16,), val, dtype=dtype)


def _sc_edge_kernel(src_hbm, dst_hbm, asrc_hbm, adst_hbm, c_hbm, h_hbm,
                    agg_hbm, sidef_hbm, sidel_hbm, ids_hbm,
                    sv, dv, exv, asv, adv, cv, rows0, rows1, acc, zrow, idsv,
                    sem0, sem1):
    wid = lax.axis_index("s") * 2 + lax.axis_index("c")
    base = wid * EW

    pltpu.sync_copy(src_hbm.at[pl.ds(base, EW)], sv)
    pltpu.sync_copy(dst_hbm.at[pl.ds(base, EW)], dv)
    pltpu.sync_copy(asrc_hbm, asv)
    pltpu.sync_copy(adst_hbm, adv)
    pltpu.sync_copy(c_hbm, cv)

    zero16 = jnp.zeros((16,), jnp.float32)
    for j in range(HD // 16):
        acc[pl.ds(j * 16, 16)] = zero16
        zrow[pl.ds(j * 16, 16)] = zero16

    cvec = cv[...]
    iota16 = lax.iota(jnp.int32, 16)

    first_d = jnp.max(plsc.load_gather(dv, [_f16(0, jnp.int32)]))
    last_d = jnp.max(plsc.load_gather(dv, [_f16(EW - 1, jnp.int32)]))
    pltpu.sync_copy(zrow, agg_hbm.at[first_d])
    pltpu.sync_copy(zrow, agg_hbm.at[last_d])

    def phase_a(i, carry):
        off = pl.multiple_of(i * 16, 16)
        s16 = sv[pl.ds(off, 16)]
        d16 = dv[pl.ds(off, 16)]
        asg = plsc.load_gather(asv, [s16])
        adg = plsc.load_gather(adv, [d16])
        s = asg + adg
        e = jnp.where(s >= 0.0, s, 0.2 * s)
        ex = jnp.exp(e - cvec)
        gidx = _f16(base + i * 16, jnp.int32) + iota16
        ex = jnp.where(gidx < E2, ex, 0.0)
        exv[pl.ds(off, 16)] = ex
        return carry

    lax.fori_loop(0, EW // 16, phase_a, 0)

    NCH = EW // CHUNK

    def _gather_desc(cc, buf, sem):
        off = pl.multiple_of(cc * CHUNK, 8)
        return pltpu.make_async_copy(
            h_hbm.at[sv.at[pl.ds(off, CHUNK)]], buf, sem)

    _gather_desc(0, rows0, sem0).start()
    _gather_desc(1, rows1, sem1).start()

    def phase_b(p, carry):
        prev, seg = carry
        for b, (rows, sem) in enumerate(((rows0, sem0), (rows1, sem1))):
            cc = p * 2 + b
            off = pl.multiple_of(cc * CHUNK, 8)
            _gather_desc(cc, rows, sem).wait()
            for j in range(CHUNK):
                jb = _f16(0, jnp.int32) + (off + j)
                exb = plsc.load_gather(exv, [jb])
                dsc = jnp.max(plsc.load_gather(dv, [jb]))
                changed = dsc != prev

                @pl.when(jnp.logical_and(changed, seg > 0))
                def _flush_interior():
                    pltpu.sync_copy(acc, agg_hbm.at[prev])

                @pl.when(jnp.logical_and(changed, seg == 0))
                def _flush_first():
                    pltpu.sync_copy(acc, sidef_hbm.at[wid])

                @pl.when(changed)
                def _clear():
                    for j2 in range(HD // 16):
                        acc[pl.ds(j2 * 16, 16)] = zero16

                for j2 in range(H // 16):
                    sl = pl.ds(j2 * 16, 16)
                    acc[sl] += exb * rows[j, sl]
                acc[pl.ds(H, 16)] += exb
                prev = dsc
                seg = seg + changed.astype(jnp.int32)

            @pl.when(cc + 2 < NCH)
            def _prefetch():
                _gather_desc(cc + 2, rows, sem).start()
        return prev, seg

    prev_f, seg_f = lax.fori_loop(0, NCH // 2, phase_b,
                                  (first_d, jnp.int32(0)))

    pltpu.sync_copy(acc, sidel_hbm.at[wid])

    @pl.when(seg_f == 0)
    def _no_first():
        pltpu.sync_copy(zrow, sidef_hbm.at[wid])

    fd_out = jnp.where(seg_f == 0, jnp.int32(SENT), first_d)
    ids = jnp.where(iota16 == 0, _f16(0, jnp.int32) + fd_out,
                    jnp.where(iota16 == 1, _f16(0, jnp.int32) + prev_f,
                              _f16(SENT, jnp.int32)))
    idsv[...] = ids
    pltpu.sync_copy(idsv, ids_hbm.at[wid])


def _sc_edge_call(src_p, dst_p, asrc, adst, cvec, h):
    mesh = plsc.VectorSubcoreMesh(core_axis_name="c", subcore_axis_name="s")
    fn = functools.partial(
        pl.kernel, mesh=mesh,
        compiler_params=pltpu.CompilerParams(needs_layout_passes=False),
        out_type=[
            jax.ShapeDtypeStruct((N, HD), jnp.float32),
            jax.ShapeDtypeStruct((NW, HD), jnp.float32),
            jax.ShapeDtypeStruct((NW, HD), jnp.float32),
            jax.ShapeDtypeStruct((NW, 16), jnp.int32),
        ],
        scratch_types=[
            pltpu.VMEM((EW,), jnp.int32),
            pltpu.VMEM((EW,), jnp.int32),
            pltpu.VMEM((EW,), jnp.float32),
            pltpu.VMEM((N,), jnp.float32),
            pltpu.VMEM((N,), jnp.float32),
            pltpu.VMEM((16,), jnp.float32),
            pltpu.VMEM((CHUNK, H), jnp.float32),
            pltpu.VMEM((CHUNK, H), jnp.float32),
            pltpu.VMEM((HD,), jnp.float32),
            pltpu.VMEM((HD,), jnp.float32),
            pltpu.VMEM((16,), jnp.int32),
            pltpu.SemaphoreType.DMA,
            pltpu.SemaphoreType.DMA,
        ],
    )(_sc_edge_kernel)
    return fn(src_p, dst_p, asrc, adst, cvec, h)


def _pre1_body(x_ref, w_ref, as_ref, ad_ref, h_ref, s_ref, d_ref, ms_ref,
               md_ref):
    i = pl.program_id(0)
    h = jnp.dot(x_ref[...], w_ref[...], preferred_element_type=jnp.float32)
    h_ref[...] = h
    s = jnp.dot(h, as_ref[...], preferred_element_type=jnp.float32)
    d = jnp.dot(h, ad_ref[...], preferred_element_type=jnp.float32)
    s_ref[...] = s
    d_ref[...] = d

    @pl.when(i == 0)
    def _init():
        ms_ref[...] = jnp.full((1, 1), -jnp.inf, jnp.float32)
        md_ref[...] = jnp.full((1, 1), -jnp.inf, jnp.float32)

    ms_ref[...] = jnp.maximum(ms_ref[...], jnp.max(s))
    md_ref[...] = jnp.maximum(md_ref[...], jnp.max(d))


def _pre1_call(x, W1, a_src, a_dst):
    grid = (N // BN,)
    return pl.pallas_call(
        _pre1_body,
        grid=grid,
        in_specs=[
            pl.BlockSpec((BN, D_IN), lambda i: (i, 0)),
            pl.BlockSpec((D_IN, H), lambda i: (0, 0)),
            pl.BlockSpec((H, 1), lambda i: (0, 0)),
            pl.BlockSpec((H, 1), lambda i: (0, 0)),
        ],
        out_specs=[
            pl.BlockSpec((BN, H), lambda i: (i, 0)),
            pl.BlockSpec((BN, 1), lambda i: (i, 0)),
            pl.BlockSpec((BN, 1), lambda i: (i, 0)),
            pl.BlockSpec((1, 1), lambda i: (0, 0)),
            pl.BlockSpec((1, 1), lambda i: (0, 0)),
        ],
        out_shape=[
            jax.ShapeDtypeStruct((N, H), jnp.float32),
            jax.ShapeDtypeStruct((N, 1), jnp.float32),
            jax.ShapeDtypeStruct((N, 1), jnp.float32),
            jax.ShapeDtypeStruct((1, 1), jnp.float32),
            jax.ShapeDtypeStruct((1, 1), jnp.float32),
        ],
    )(x, W1, a_src, a_dst)


def _fix_body(agg_ref, sf_ref, sl_ref, ids_ref, b_ref, out_ref, sm_ref,
              sq_ref):
    i = pl.program_id(0)
    rowid = i * BN + lax.broadcasted_iota(jnp.int32, (BN, 1), 0)
    ids = ids_ref[...]
    idf = ids[:, 0].reshape(1, NW)
    idl = ids[:, 1].reshape(1, NW)
    eqf = (rowid == idf).astype(jnp.float32)
    eql = (rowid == idl).astype(jnp.float32)
    fix = (agg_ref[...]
           + jnp.dot(eqf, sf_ref[...], preferred_element_type=jnp.float32)
           + jnp.dot(eql, sl_ref[...], preferred_element_type=jnp.float32))
    row = fix[:, :H] / (fix[:, H:H + 1] + 1e-16) + b_ref[...]
    out_ref[...] = row

    @pl.when(i == 0)
    def _init():
        sm_ref[...] = jnp.zeros((1, H), jnp.float32)
        sq_ref[...] = jnp.zeros((1, H), jnp.float32)

    sm_ref[...] += jnp.sum(row, axis=0, keepdims=True)
    sq_ref[...] += jnp.sum(row * row, axis=0, keepdims=True)


def _fix_call(agg, sidef, sidel, ids, b):
    grid = (N // BN,)
    return pl.pallas_call(
        _fix_body,
        grid=grid,
        in_specs=[
            pl.BlockSpec((BN, HD), lambda i: (i, 0)),
            pl.BlockSpec((NW, HD), lambda i: (0, 0)),
            pl.BlockSpec((NW, HD), lambda i: (0, 0)),
            pl.BlockSpec((NW, 16), lambda i: (0, 0)),
            pl.BlockSpec((1, H), lambda i: (0, 0)),
        ],
        out_specs=[
            pl.BlockSpec((BN, H), lambda i: (i, 0)),
            pl.BlockSpec((1, H), lambda i: (0, 0)),
            pl.BlockSpec((1, H), lambda i: (0, 0)),
        ],
        out_shape=[
            jax.ShapeDtypeStruct((N, H), jnp.float32),
            jax.ShapeDtypeStruct((1, H), jnp.float32),
            jax.ShapeDtypeStruct((1, H), jnp.float32),
        ],
    )(agg, sidef, sidel, ids, b)


def _norm_pre2_body(x_ref, sm_ref, sq_ref, g_ref, be_ref, bat_ref, w_ref,
                    as_ref, ad_ref, hr_ref, h2_ref, s_ref, d_ref, ms_ref,
                    md_ref, pool_ref, cnt_ref):
    i = pl.program_id(0)
    mu = sm_ref[...] / N
    var = sq_ref[...] / N - mu * mu
    xn = (x_ref[...] - mu) / jnp.sqrt(var + 1e-5) * g_ref[...] + be_ref[...]
    hr = jnp.maximum(xn, 0.0)
    hr_ref[...] = hr
    h2 = jnp.dot(hr, w_ref[...], preferred_element_type=jnp.float32)
    h2_ref[...] = h2
    s = jnp.dot(h2, as_ref[...], preferred_element_type=jnp.float32)
    d = jnp.dot(h2, ad_ref[...], preferred_element_type=jnp.float32)
    s_ref[...] = s
    d_ref[...] = d

    grp = lax.broadcasted_iota(jnp.int32, (NG, BN), 0)
    bo = (grp == bat_ref[...].reshape(1, BN)).astype(jnp.float32)

    @pl.when(i == 0)
    def _init():
        ms_ref[...] = jnp.full((1, 1), -jnp.inf, jnp.float32)
        md_ref[...] = jnp.full((1, 1), -jnp.inf, jnp.float32)
        pool_ref[...] = jnp.zeros((NG, H), jnp.float32)
        cnt_ref[...] = jnp.zeros((NG, 1), jnp.float32)

    ms_ref[...] = jnp.maximum(ms_ref[...], jnp.max(s))
    md_ref[...] = jnp.maximum(md_ref[...], jnp.max(d))
    pool_ref[...] += jnp.dot(bo, hr, preferred_element_type=jnp.float32)
    cnt_ref[...] += jnp.sum(bo, axis=1, keepdims=True)


def _norm_pre2_call(x, sm, sq, g, be, bat, W2, a_src, a_dst):
    grid = (N // BN,)
    return pl.pallas_call(
        _norm_pre2_body,
        grid=grid,
        in_specs=[
            pl.BlockSpec((BN, H), lambda i: (i, 0)),
            pl.BlockSpec((1, H), lambda i: (0, 0)),
            pl.BlockSpec((1, H), lambda i: (0, 0)),
            pl.BlockSpec((1, H), lambda i: (0, 0)),
            pl.BlockSpec((1, H), lambda i: (0, 0)),
            pl.BlockSpec((BN, 1), lambda i: (i, 0)),
            pl.BlockSpec((H, H), lambda i: (0, 0)),
            pl.BlockSpec((H, 1), lambda i: (0, 0)),
            pl.BlockSpec((H, 1), lambda i: (0, 0)),
        ],
        out_specs=[
            pl.BlockSpec((BN, H), lambda i: (i, 0)),
            pl.BlockSpec((BN, H), lambda i: (i, 0)),
            pl.BlockSpec((BN, 1), lambda i: (i, 0)),
            pl.BlockSpec((BN, 1), lambda i: (i, 0)),
            pl.BlockSpec((1, 1), lambda i: (0, 0)),
            pl.BlockSpec((1, 1), lambda i: (0, 0)),
            pl.BlockSpec((NG, H), lambda i: (0, 0)),
            pl.BlockSpec((NG, 1), lambda i: (0, 0)),
        ],
        out_shape=[
            jax.ShapeDtypeStruct((N, H), jnp.float32),
            jax.ShapeDtypeStruct((N, H), jnp.float32),
            jax.ShapeDtypeStruct((N, 1), jnp.float32),
            jax.ShapeDtypeStruct((N, 1), jnp.float32),
            jax.ShapeDtypeStruct((1, 1), jnp.float32),
            jax.ShapeDtypeStruct((1, 1), jnp.float32),
            jax.ShapeDtypeStruct((NG, H), jnp.float32),
            jax.ShapeDtypeStruct((NG, 1), jnp.float32),
        ],
    )(x, sm, sq, g, be, bat, W2, a_src, a_dst)


def _final_body(x_ref, sm_ref, sq_ref, g_ref, be_ref, bat_ref, p1_ref,
                cnt_ref, fcw_ref, fcb_ref, fc1w_ref, fc1b_ref, p2_ref,
                out_ref):
    i = pl.program_id(0)
    mu = sm_ref[...] / N
    var = sq_ref[...] / N - mu * mu
    xn = (x_ref[...] - mu) / jnp.sqrt(var + 1e-5) * g_ref[...] + be_ref[...]
    hr = jnp.maximum(xn, 0.0)

    grp = lax.broadcasted_iota(jnp.int32, (NG, BN), 0)
    bo = (grp == bat_ref[...].reshape(1, BN)).astype(jnp.float32)

    @pl.when(i == 0)
    def _init():
        p2_ref[...] = jnp.zeros((NG, H), jnp.float32)

    p2_ref[...] += jnp.dot(bo, hr, preferred_element_type=jnp.float32)

    @pl.when(i == (N // BN) - 1)
    def _head():
        cnt = jnp.maximum(cnt_ref[...], 1.0)
        z = p1_ref[...] / cnt + p2_ref[...] / cnt
        z = jnp.maximum(
            jnp.dot(z, fcw_ref[...], preferred_element_type=jnp.float32)
            + fcb_ref[...], 0.0)
        z = jnp.dot(z, fc1w_ref[...],
                    preferred_element_type=jnp.float32) + fc1b_ref[...]
        m = jnp.max(z, axis=1, keepdims=True)
        lse = m + jnp.log(jnp.sum(jnp.exp(z - m), axis=1, keepdims=True))
        out_ref[...] = z - lse


def _final_call(x, sm, sq, g, be, bat, pool1, cnt, fcw, fcb, fc1w, fc1b):
    grid = (N // BN,)
    return pl.pallas_call(
        _final_body,
        grid=grid,
        in_specs=[
            pl.BlockSpec((BN, H), lambda i: (i, 0)),
            pl.BlockSpec((1, H), lambda i: (0, 0)),
            pl.BlockSpec((1, H), lambda i: (0, 0)),
            pl.BlockSpec((1, H), lambda i: (0, 0)),
            pl.BlockSpec((1, H), lambda i: (0, 0)),
            pl.BlockSpec((BN, 1), lambda i: (i, 0)),
            pl.BlockSpec((NG, H), lambda i: (0, 0)),
            pl.BlockSpec((NG, 1), lambda i: (0, 0)),
            pl.BlockSpec((H, FC), lambda i: (0, 0)),
            pl.BlockSpec((1, FC), lambda i: (0, 0)),
            pl.BlockSpec((FC, OUT), lambda i: (0, 0)),
            pl.BlockSpec((1, OUT), lambda i: (0, 0)),
        ],
        out_specs=[
            pl.BlockSpec((NG, H), lambda i: (0, 0)),
            pl.BlockSpec((NG, OUT), lambda i: (0, 0)),
        ],
        out_shape=[
            jax.ShapeDtypeStruct((NG, H), jnp.float32),
            jax.ShapeDtypeStruct((NG, OUT), jnp.float32),
        ],
    )(x, sm, sq, g, be, bat, pool1, cnt, fcw, fcb, fc1w, fc1b)


def _shift_const(ms, md):
    c = ms[0, 0] + md[0, 0]
    c = jnp.where(c > 0.0, c, 0.2 * c)
    return jnp.full((16,), c, jnp.float32)


def kernel(x, edge_index, batch, W1, a_src1, a_dst1, b1, g1, be1,
           W2, a_src2, a_dst2, b2, g2, be2, fc_w, fc_b, fc1_w, fc1_b):
    loop = jnp.arange(N, dtype=jnp.int32)
    src = jnp.concatenate([edge_index[0].astype(jnp.int32), loop])
    dst = jnp.concatenate([edge_index[1].astype(jnp.int32), loop])
    order = jnp.argsort(dst)
    src_s = src[order]
    dst_s = dst[order]
    npad = EPAD - E2
    src_p = jnp.concatenate([src_s, jnp.zeros((npad,), jnp.int32)])
    dst_p = jnp.concatenate([dst_s, jnp.full((npad,), N - 1, jnp.int32)])
    bat = batch.astype(jnp.int32).reshape(N, 1)

    h1, s1, d1, ms1, md1 = _pre1_call(
        x, W1, a_src1.reshape(H, 1), a_dst1.reshape(H, 1))
    c1 = _shift_const(ms1, md1)
    agg1, sf1, sl1, ids1 = _sc_edge_call(
        src_p, dst_p, s1.reshape(N), d1.reshape(N), c1, h1)
    out1, sm1, sq1 = _fix_call(agg1, sf1, sl1, ids1, b1.reshape(1, H))

    hr1, h2, s2, d2, ms2, md2, pool1, cnt = _norm_pre2_call(
        out1, sm1, sq1, g1.reshape(1, H), be1.reshape(1, H), bat, W2,
        a_src2.reshape(H, 1), a_dst2.reshape(H, 1))

    c2 = _shift_const(ms2, md2)
    agg2, sf2, sl2, ids2 = _sc_edge_call(
        src_p, dst_p, s2.reshape(N), d2.reshape(N), c2, h2)
    out2, sm2, sq2 = _fix_call(agg2, sf2, sl2, ids2, b2.reshape(1, H))

    _, out = _final_call(
        out2, sm2, sq2, g2.reshape(1, H), be2.reshape(1, H), bat, pool1,
        cnt, fc_w, fc_b.reshape(1, FC), fc1_w, fc1_b.reshape(1, OUT))
    return out
